# baseline (speedup 1.0000x reference)
import jax
import jax.numpy as jnp
from jax import lax
from jax.experimental import pallas as pl
from jax.experimental.pallas import tpu as pltpu
from jax.experimental.pallas import tpu_sc as plsc

N_TYPES = 1000
WINDOW = 1
DIM = (1 + 2 * WINDOW) * N_TYPES
B, L = 1024, 20
ROW = L * DIM
LP = 32

NC, NS, NL = 2, 16, 16
NW = NC * NS
BPW = B // NW


def _bow_sc(txt_hbm, vals_hbm, zeros_hbm, out_hbm, txt_v, vals_v,
            buf0, buf1, sem0, sem1, insem):
    wid = lax.axis_index("s") * NC + lax.axis_index("c")
    base = wid * (BPW * LP)

    pltpu.sync_copy(txt_hbm.at[pl.ds(base, BPW * LP)], txt_v)
    pltpu.sync_copy(vals_hbm.at[pl.ds(base, BPW * LP)], vals_v)

    c0 = pltpu.async_copy(zeros_hbm, buf0, insem)
    c1 = pltpu.async_copy(zeros_hbm, buf1, sem1)
    c0.wait()
    c1.wait()

    bufs = (buf0, buf1)
    sems = (sem0, sem1)
    iota = lax.iota(jnp.int32, NL)
    fzero = jnp.zeros((NL,), jnp.float32)

    pend = [None, None]
    dirty = [None, None]

    for b in range(BPW):
        p = b % 2
        buf = bufs[p]

        if pend[p] is not None:
            pend[p].wait()
            for rows, cols, msk in dirty[p]:
                plsc.store_scatter(buf, [rows, cols], fzero, mask=msk)

        writes = []
        for half in range(2):
            n = iota + (NL * half)
            off = b * LP + NL * half
            tok = txt_v[pl.ds(off, NL)]
            val = vals_v[pl.ds(off, NL)]
            h = lax.rem(tok, N_TYPES)
            in_seq = n < L
            rows_l = jnp.where(n == 0, L - 1, n - 1)
            trip = ((rows_l, h, in_seq),
                    (n, N_TYPES + h, in_seq),
                    (n + 1, 2 * N_TYPES + h, n < (L - 1)))
            for rows, cols, msk in trip:
                plsc.store_scatter(buf, [rows, cols], val, mask=msk)
                writes.append((rows, cols, msk))

        dirty[p] = writes
        pend[p] = pltpu.async_copy(buf, out_hbm.at[wid * BPW + b], sems[p])

    pend[0].wait()
    pend[1].wait()


@jax.jit
def _bow(txt, vals):
    txt_p = jnp.pad(txt, ((0, 0), (0, LP - L))).reshape(-1)
    vals_p = jnp.pad(vals, ((0, 0), (0, LP - L))).reshape(-1)
    zrow = jnp.zeros((L, DIM), jnp.float32)
    mesh = plsc.VectorSubcoreMesh(core_axis_name="c", subcore_axis_name="s")
    return pl.kernel(
        _bow_sc,
        mesh=mesh,
        compiler_params=pltpu.CompilerParams(
            needs_layout_passes=False, use_tc_tiling_on_sc=False),
        out_type=jax.ShapeDtypeStruct((B, L, DIM), jnp.float32),
        scratch_types=[
            pltpu.VMEM((BPW * LP,), jnp.int32),
            pltpu.VMEM((BPW * LP,), jnp.float32),
            pltpu.VMEM((L, DIM), jnp.float32),
            pltpu.VMEM((L, DIM), jnp.float32),
            pltpu.SemaphoreType.DMA,
            pltpu.SemaphoreType.DMA,
            pltpu.SemaphoreType.DMA,
        ],
    )(txt_p, vals_p, zrow)


def kernel(txt, vals):
    return _bow(txt, vals)


# speedup vs baseline: 2.0127x; 2.0127x over previous
import jax
import jax.numpy as jnp
from jax import lax
from jax.experimental import pallas as pl
from jax.experimental.pallas import tpu as pltpu
from jax.experimental.pallas import tpu_sc as plsc

N_TYPES = 1000
WINDOW = 1
DIM = (1 + 2 * WINDOW) * N_TYPES
B, L = 1024, 20
LP = 32

NC, NS, NL = 2, 16, 16
NW = NC * NS
BPW = B // NW


def _bow_sc(txt_hbm, vals_hbm, zeros_hbm, out_hbm, txt_v, vals_v,
            buf, sem, insem):
    wid = lax.axis_index("s") * NC + lax.axis_index("c")
    base = wid * (BPW * LP)

    pltpu.sync_copy(txt_hbm.at[pl.ds(base, BPW * LP)], txt_v)
    pltpu.sync_copy(vals_hbm.at[pl.ds(base, BPW * LP)], vals_v)

    pltpu.async_copy(zeros_hbm, buf, insem).wait()

    iota = lax.iota(jnp.int32, NL)
    fzero = jnp.zeros((NL,), jnp.float32)

    pend = None
    dirty = None

    for b in range(BPW):
        if pend is not None:
            pend.wait()
            for rows, cols, msk in dirty:
                plsc.store_scatter(buf, [rows, cols], fzero, mask=msk)

        writes = []
        for half in range(2):
            n = iota + (NL * half)
            off = b * LP + NL * half
            tok = txt_v[pl.ds(off, NL)]
            val = vals_v[pl.ds(off, NL)]
            h = lax.rem(tok, N_TYPES)
            in_seq = n < L
            rows_l = jnp.where(n == 0, L - 1, n - 1)
            trip = ((rows_l, h, in_seq),
                    (n, N_TYPES + h, in_seq),
                    (n + 1, 2 * N_TYPES + h, n < (L - 1)))
            for rows, cols, msk in trip:
                plsc.store_scatter(buf, [rows, cols], val, mask=msk)
                writes.append((rows, cols, msk))

        dirty = writes
        pend = pltpu.async_copy(buf, out_hbm.at[wid * BPW + b], sem)

    pend.wait()


@jax.jit
def _bow(txt, vals):
    txt_p = jnp.pad(txt, ((0, 0), (0, LP - L))).reshape(-1)
    vals_p = jnp.pad(vals, ((0, 0), (0, LP - L))).reshape(-1)
    zrow = jnp.zeros((L, DIM), jnp.float32)
    mesh = plsc.VectorSubcoreMesh(core_axis_name="c", subcore_axis_name="s")
    return pl.kernel(
        _bow_sc,
        mesh=mesh,
        compiler_params=pltpu.CompilerParams(
            needs_layout_passes=False, use_tc_tiling_on_sc=True),
        out_type=jax.ShapeDtypeStruct((B, L, DIM), jnp.float32),
        scratch_types=[
            pltpu.VMEM((BPW * LP,), jnp.int32),
            pltpu.VMEM((BPW * LP,), jnp.float32),
            pltpu.VMEM((L, DIM), jnp.float32),
            pltpu.SemaphoreType.DMA,
            pltpu.SemaphoreType.DMA,
        ],
    )(txt_p, vals_p, zrow)


def kernel(txt, vals):
    return _bow(txt, vals)


# speedup vs baseline: 2.3100x; 1.1477x over previous
import jax
import jax.numpy as jnp
from jax import lax
from jax.experimental import pallas as pl
from jax.experimental.pallas import tpu as pltpu
from jax.experimental.pallas import tpu_sc as plsc

N_TYPES = 1000
WINDOW = 1
NBAND = 1 + 2 * WINDOW
DIM = NBAND * N_TYPES
B, L = 1024, 20

NC, NS, NL = 2, 16, 16
NW = NC * NS
NG = B // NL

HW = 40
NWIN = N_TYPES // HW
NCHUNK = L * NBAND * NWIN
KMAX = 48


def _chunk_params(wid, k):
    c = jnp.minimum(wid + NW * k, NCHUNK - 1)
    m = c // (NBAND * NWIN)
    r = c - m * (NBAND * NWIN)
    band = r // NWIN
    wi = r - band * NWIN
    lo = wi * HW
    v0 = band * N_TYPES + lo
    n = jnp.where(band == 0, lax.rem(m + 1, L),
                  jnp.where(band == 1, m, m - 1))
    valid = jnp.logical_or(band != 2, m > 0)
    n = jnp.maximum(n, 0)
    return m, v0, n, lo, valid


def _scan_scatter(txt_v, vals_v, buf, n, lo, valid, iota, write_vals, fzero):
    @pl.loop(0, NG, unroll=2)
    def body(g):
        off = n * B + g * NL
        tok = txt_v[pl.ds(off, NL)]
        h = lax.rem(tok, N_TYPES)
        hr = h - lo
        msk = jnp.logical_and(
            jnp.logical_and(hr >= 0, hr < HW), valid)
        jv = iota + g * NL
        if write_vals:
            val = vals_v[pl.ds(off, NL)]
        else:
            val = fzero
        plsc.store_scatter(buf, [hr, jv], val, mask=msk)


def _bow_sc(txt_hbm, vals_hbm, zeros_hbm, out_hbm, txt_v, vals_v,
            buf0, buf1, sem0, sem1, insem):
    wid = lax.axis_index("s") * NC + lax.axis_index("c")

    pltpu.sync_copy(txt_hbm, txt_v)
    pltpu.sync_copy(vals_hbm, vals_v)

    c0 = pltpu.async_copy(zeros_hbm, buf0, insem)
    c1 = pltpu.async_copy(zeros_hbm, buf1, sem1)
    c0.wait()
    c1.wait()

    bufs = (buf0, buf1)
    sems = (sem0, sem1)
    iota = lax.iota(jnp.int32, NL)
    fzero = jnp.zeros((NL,), jnp.float32)

    def fill_and_send(k, p):
        m, v0, n, lo, valid = _chunk_params(wid, k)
        _scan_scatter(txt_v, vals_v, bufs[p], n, lo, valid, iota, True, fzero)
        pltpu.async_copy(bufs[p], out_hbm.at[m, pl.ds(v0, HW)], sems[p])

    def drain(k, p):
        m, v0, _, _, _ = _chunk_params(wid, k)
        pltpu.make_async_copy(
            bufs[p], out_hbm.at[m, pl.ds(v0, HW)], sems[p]).wait()

    fill_and_send(0, 0)
    fill_and_send(1, 1)

    @pl.loop(2, KMAX, step=2)
    def step(k):
        for p in range(2):
            kk = k + p
            drain(kk - 2, p)
            pn_m, pv0, pn, plo, pvalid = _chunk_params(wid, kk - 2)
            _scan_scatter(txt_v, vals_v, bufs[p], pn, plo, pvalid, iota,
                          False, fzero)
            fill_and_send(kk, p)

    drain(KMAX - 2, 0)
    drain(KMAX - 1, 1)


@jax.jit
def _bow(txt, vals):
    txt_t = txt.T.reshape(-1)
    vals_t = vals.T.reshape(-1)
    zchunk = jnp.zeros((HW, B), jnp.float32)
    mesh = plsc.VectorSubcoreMesh(core_axis_name="c", subcore_axis_name="s")
    out = pl.kernel(
        _bow_sc,
        mesh=mesh,
        compiler_params=pltpu.CompilerParams(
            needs_layout_passes=False, use_tc_tiling_on_sc=True),
        out_type=jax.ShapeDtypeStruct((L, DIM, B), jnp.float32),
        scratch_types=[
            pltpu.VMEM((L * B,), jnp.int32),
            pltpu.VMEM((L * B,), jnp.float32),
            pltpu.VMEM((HW, B), jnp.float32),
            pltpu.VMEM((HW, B), jnp.float32),
            pltpu.SemaphoreType.DMA,
            pltpu.SemaphoreType.DMA,
            pltpu.SemaphoreType.DMA,
        ],
    )(txt_t, vals_t, zchunk)
    return out.transpose(2, 0, 1)


def kernel(txt, vals):
    return _bow(txt, vals)


# speedup vs baseline: 4.8473x; 2.0984x over previous
import jax
import jax.numpy as jnp
from jax import lax
from jax.experimental import pallas as pl
from jax.experimental.pallas import tpu as pltpu
from jax.experimental.pallas import tpu_sc as plsc

N_TYPES = 1000
WINDOW = 1
NBAND = 1 + 2 * WINDOW
DIM = NBAND * N_TYPES
B, L = 1024, 20

NC, NS, NL = 2, 16, 16
NW = NC * NS
NG = B // NL

HW = 40
NWIN = N_TYPES // HW
NCHUNK = L * NBAND * NWIN
KMAX = 48


def _chunk_params(wid, k):
    c = jnp.minimum(wid + NW * k, NCHUNK - 1)
    m = c // (NBAND * NWIN)
    r = c - m * (NBAND * NWIN)
    band = r // NWIN
    wi = r - band * NWIN
    lo = wi * HW
    v0 = band * N_TYPES + lo
    n = jnp.where(band == 0, lax.rem(m + 1, L),
                  jnp.where(band == 1, m, m - 1))
    valid = jnp.logical_or(band != 2, m > 0)
    n = jnp.maximum(n, 0)
    lo = jnp.where(valid, lo, -2 * N_TYPES)
    return m, v0, n, lo


def _scan_scatter(h_v, vals_v, buf, n, lo, iota, write_vals, fzero):
    @pl.loop(0, NG, init_carry=(n * B, iota), unroll=8)
    def body(g, carry):
        off, jv = carry
        hr = h_v[pl.ds(off, NL)] - lo
        msk = lax.bitcast_convert_type(hr, jnp.uint32) < jnp.uint32(HW)
        if write_vals:
            val = vals_v[pl.ds(off, NL)]
        else:
            val = fzero
        plsc.store_scatter(buf, [hr, jv], val, mask=msk)
        return off + NL, jv + NL


def _bow_sc(txt_hbm, vals_hbm, zeros_hbm, out_hbm, h_v, vals_v,
            buf0, buf1, sem0, sem1, insem):
    wid = lax.axis_index("s") * NC + lax.axis_index("c")

    pltpu.sync_copy(txt_hbm, h_v)
    pltpu.sync_copy(vals_hbm, vals_v)

    c0 = pltpu.async_copy(zeros_hbm, buf0, insem)
    c1 = pltpu.async_copy(zeros_hbm, buf1, sem1)

    @pl.loop(0, L * NG, unroll=8)
    def prehash(g):
        sl = pl.ds(g * NL, NL)
        h_v[sl] = lax.rem(h_v[sl], N_TYPES)

    c0.wait()
    c1.wait()

    bufs = (buf0, buf1)
    sems = (sem0, sem1)
    iota = lax.iota(jnp.int32, NL)
    fzero = jnp.zeros((NL,), jnp.float32)

    def fill_and_send(k, p):
        m, v0, n, lo = _chunk_params(wid, k)
        _scan_scatter(h_v, vals_v, bufs[p], n, lo, iota, True, fzero)
        pltpu.async_copy(bufs[p], out_hbm.at[m, pl.ds(v0, HW)], sems[p])

    def drain(k, p):
        m, v0, _, _ = _chunk_params(wid, k)
        pltpu.make_async_copy(
            bufs[p], out_hbm.at[m, pl.ds(v0, HW)], sems[p]).wait()

    fill_and_send(0, 0)
    fill_and_send(1, 1)

    @pl.loop(2, KMAX, step=2)
    def step(k):
        for p in range(2):
            kk = k + p
            drain(kk - 2, p)
            _, _, pn, plo = _chunk_params(wid, kk - 2)
            _scan_scatter(h_v, vals_v, bufs[p], pn, plo, iota, False, fzero)
            fill_and_send(kk, p)

    drain(KMAX - 2, 0)
    drain(KMAX - 1, 1)


@jax.jit
def _bow(txt, vals):
    txt_t = txt.T.reshape(-1)
    vals_t = vals.T.reshape(-1)
    zchunk = jnp.zeros((HW, B), jnp.float32)
    mesh = plsc.VectorSubcoreMesh(core_axis_name="c", subcore_axis_name="s")
    out = pl.kernel(
        _bow_sc,
        mesh=mesh,
        compiler_params=pltpu.CompilerParams(
            needs_layout_passes=False, use_tc_tiling_on_sc=True),
        out_type=jax.ShapeDtypeStruct((L, DIM, B), jnp.float32),
        scratch_types=[
            pltpu.VMEM((L * B,), jnp.int32),
            pltpu.VMEM((L * B,), jnp.float32),
            pltpu.VMEM((HW, B), jnp.float32),
            pltpu.VMEM((HW, B), jnp.float32),
            pltpu.SemaphoreType.DMA,
            pltpu.SemaphoreType.DMA,
            pltpu.SemaphoreType.DMA,
        ],
    )(txt_t, vals_t, zchunk)
    return out.transpose(2, 0, 1)


def kernel(txt, vals):
    return _bow(txt, vals)


# speedup vs baseline: 4.8616x; 1.0030x over previous
import jax
import jax.numpy as jnp
from jax import lax
from jax.experimental import pallas as pl
from jax.experimental.pallas import tpu as pltpu
from jax.experimental.pallas import tpu_sc as plsc

N_TYPES = 1000
WINDOW = 1
NBAND = 1 + 2 * WINDOW
DIM = NBAND * N_TYPES
B, L = 1024, 20

NC, NS, NL = 2, 16, 16
NW = NC * NS
NG = B // NL

HW = 40
NWIN = N_TYPES // HW
NCHUNK = L * NBAND * NWIN
KMAX = 48


def _chunk_params(wid, k):
    c = jnp.minimum(wid + NW * k, NCHUNK - 1)
    m = c // (NBAND * NWIN)
    r = c - m * (NBAND * NWIN)
    band = r // NWIN
    wi = r - band * NWIN
    lo = wi * HW
    v0 = band * N_TYPES + lo
    n = jnp.where(band == 0, lax.rem(m + 1, L),
                  jnp.where(band == 1, m, m - 1))
    valid = jnp.logical_or(band != 2, m > 0)
    n = jnp.maximum(n, 0)
    lo = jnp.where(valid, lo, -2 * N_TYPES)
    return m, v0, n, lo


def _scan_scatter(h_v, vals_v, buf, n, lo, iota, write_vals, fzero):
    @pl.loop(0, NG, init_carry=(n * B, iota), unroll=8)
    def body(g, carry):
        off, jv = carry
        hr = h_v[pl.ds(off, NL)] - lo
        msk = lax.bitcast_convert_type(hr, jnp.uint32) < jnp.uint32(HW)
        if write_vals:
            val = vals_v[pl.ds(off, NL)]
        else:
            val = fzero
        plsc.store_scatter(buf, [hr, jv], val, mask=msk)
        return off + NL, jv + NL


def _reset_and_scan(h_v, vals_v, buf, pn, plo, n, lo, iota, fzero):
    @pl.loop(0, NG, init_carry=(pn * B, n * B, iota), unroll=8)
    def body(g, carry):
        poff, off, jv = carry
        phr = h_v[pl.ds(poff, NL)] - plo
        pmsk = lax.bitcast_convert_type(phr, jnp.uint32) < jnp.uint32(HW)
        plsc.store_scatter(buf, [phr, jv], fzero, mask=pmsk)
        hr = h_v[pl.ds(off, NL)] - lo
        msk = lax.bitcast_convert_type(hr, jnp.uint32) < jnp.uint32(HW)
        val = vals_v[pl.ds(off, NL)]
        plsc.store_scatter(buf, [hr, jv], val, mask=msk)
        return poff + NL, off + NL, jv + NL


def _bow_sc(txt_hbm, vals_hbm, zeros_hbm, out_hbm, h_v, vals_v,
            buf0, buf1, sem0, sem1, insem):
    wid = lax.axis_index("s") * NC + lax.axis_index("c")

    pltpu.sync_copy(txt_hbm, h_v)
    pltpu.sync_copy(vals_hbm, vals_v)

    c0 = pltpu.async_copy(zeros_hbm, buf0, insem)
    c1 = pltpu.async_copy(zeros_hbm, buf1, sem1)

    @pl.loop(0, L * NG, unroll=8)
    def prehash(g):
        sl = pl.ds(g * NL, NL)
        h_v[sl] = lax.rem(h_v[sl], N_TYPES)

    c0.wait()
    c1.wait()

    bufs = (buf0, buf1)
    sems = (sem0, sem1)
    iota = lax.iota(jnp.int32, NL)
    fzero = jnp.zeros((NL,), jnp.float32)

    def fill_and_send(k, p):
        m, v0, n, lo = _chunk_params(wid, k)
        _scan_scatter(h_v, vals_v, bufs[p], n, lo, iota, True, fzero)
        pltpu.async_copy(bufs[p], out_hbm.at[m, pl.ds(v0, HW)], sems[p])

    def drain(k, p):
        m, v0, _, _ = _chunk_params(wid, k)
        pltpu.make_async_copy(
            bufs[p], out_hbm.at[m, pl.ds(v0, HW)], sems[p]).wait()

    fill_and_send(0, 0)
    fill_and_send(1, 1)

    @pl.loop(2, KMAX, step=2)
    def step(k):
        for p in range(2):
            kk = k + p
            drain(kk - 2, p)
            _, _, pn, plo = _chunk_params(wid, kk - 2)
            m, v0, n, lo = _chunk_params(wid, kk)
            _reset_and_scan(h_v, vals_v, bufs[p], pn, plo, n, lo, iota,
                            fzero)
            pltpu.async_copy(bufs[p], out_hbm.at[m, pl.ds(v0, HW)], sems[p])

    drain(KMAX - 2, 0)
    drain(KMAX - 1, 1)


@jax.jit
def _bow(txt, vals):
    txt_t = txt.T.reshape(-1)
    vals_t = vals.T.reshape(-1)
    zchunk = jnp.zeros((HW, B), jnp.float32)
    mesh = plsc.VectorSubcoreMesh(core_axis_name="c", subcore_axis_name="s")
    out = pl.kernel(
        _bow_sc,
        mesh=mesh,
        compiler_params=pltpu.CompilerParams(
            needs_layout_passes=False, use_tc_tiling_on_sc=True),
        out_type=jax.ShapeDtypeStruct((L, DIM, B), jnp.float32),
        scratch_types=[
            pltpu.VMEM((L * B,), jnp.int32),
            pltpu.VMEM((L * B,), jnp.float32),
            pltpu.VMEM((HW, B), jnp.float32),
            pltpu.VMEM((HW, B), jnp.float32),
            pltpu.SemaphoreType.DMA,
            pltpu.SemaphoreType.DMA,
            pltpu.SemaphoreType.DMA,
        ],
    )(txt_t, vals_t, zchunk)
    return out.transpose(2, 0, 1)


def kernel(txt, vals):
    return _bow(txt, vals)


# speedup vs baseline: 4.8713x; 1.0020x over previous
import jax
import jax.numpy as jnp
from jax import lax
from jax.experimental import pallas as pl
from jax.experimental.pallas import tpu as pltpu
from jax.experimental.pallas import tpu_sc as plsc

N_TYPES = 1000
WINDOW = 1
NBAND = 1 + 2 * WINDOW
DIM = NBAND * N_TYPES
B, L = 1024, 20

NC, NS, NL = 2, 16, 16
NW = NC * NS
NG = B // NL

HW = 40
NWIN = N_TYPES // HW
NCHUNK = L * NBAND * NWIN
KMAX = 48


def _chunk_params(wid, k):
    c = jnp.minimum(wid + NW * k, NCHUNK - 1)
    m = c // (NBAND * NWIN)
    r = c - m * (NBAND * NWIN)
    band = r // NWIN
    wi = r - band * NWIN
    lo = wi * HW
    v0 = band * N_TYPES + lo
    n = jnp.where(band == 0, lax.rem(m + 1, L),
                  jnp.where(band == 1, m, m - 1))
    valid = jnp.logical_or(band != 2, m > 0)
    n = jnp.maximum(n, 0)
    lo = jnp.where(valid, lo, -2 * N_TYPES)
    return m, v0, n, lo


def _scan_scatter(h_v, vals_v, buf, n, lo, iota, write_vals, fzero):
    @pl.loop(0, NG, init_carry=(n * B, iota), unroll=8)
    def body(g, carry):
        off, jv = carry
        hr = h_v[pl.ds(off, NL)] - lo
        msk = lax.bitcast_convert_type(hr, jnp.uint32) < jnp.uint32(HW)
        if write_vals:
            val = vals_v[pl.ds(off, NL)]
        else:
            val = fzero
        plsc.store_scatter(buf, [hr, jv], val, mask=msk)
        return off + NL, jv + NL


def _reset_and_scan(h_v, vals_v, buf, pn, plo, n, lo, iota, fzero):
    @pl.loop(0, NG, init_carry=(pn * B, n * B, iota), unroll=8)
    def body(g, carry):
        poff, off, jv = carry
        phr = h_v[pl.ds(poff, NL)] - plo
        pmsk = lax.bitcast_convert_type(phr, jnp.uint32) < jnp.uint32(HW)
        plsc.store_scatter(buf, [phr, jv], fzero, mask=pmsk)
        hr = h_v[pl.ds(off, NL)] - lo
        msk = lax.bitcast_convert_type(hr, jnp.uint32) < jnp.uint32(HW)
        val = vals_v[pl.ds(off, NL)]
        plsc.store_scatter(buf, [hr, jv], val, mask=msk)
        return poff + NL, off + NL, jv + NL


def _bow_sc(txt_hbm, vals_hbm, zeros_hbm, out_hbm, h_v, vals_v,
            buf0, buf1, sem0, sem1, insem):
    wid = lax.axis_index("s") * NC + lax.axis_index("c")

    pltpu.sync_copy(txt_hbm, h_v)
    pltpu.sync_copy(vals_hbm, vals_v)

    c0 = pltpu.async_copy(zeros_hbm, buf0, insem)
    c1 = pltpu.async_copy(zeros_hbm, buf1, sem1)

    @pl.loop(0, L * NG, unroll=8)
    def prehash(g):
        sl = pl.ds(g * NL, NL)
        h_v[sl] = lax.rem(h_v[sl], N_TYPES)

    c0.wait()
    c1.wait()

    bufs = (buf0, buf1)
    sems = (sem0, sem1)
    iota = lax.iota(jnp.int32, NL)
    fzero = jnp.zeros((NL,), jnp.float32)

    def fill_and_send(k, p):
        m, v0, n, lo = _chunk_params(wid, k)
        _scan_scatter(h_v, vals_v, bufs[p], n, lo, iota, True, fzero)
        pltpu.async_copy(bufs[p], out_hbm.at[m, pl.ds(v0, HW)], sems[p])
        return n, lo

    def drain(p):
        pltpu.make_async_copy(
            bufs[p], out_hbm.at[0, pl.ds(0, HW)], sems[p]).wait()

    n0, lo0 = fill_and_send(0, 0)
    n1, lo1 = fill_and_send(1, 1)

    @pl.loop(2, KMAX, step=2, init_carry=(n0, lo0, n1, lo1))
    def step(k, carry):
        pn0, plo0, pn1, plo1 = carry
        prev = [(pn0, plo0), (pn1, plo1)]
        cur = [None, None]
        for p in range(2):
            pn, plo = prev[p]
            drain(p)
            m, v0, n, lo = _chunk_params(wid, k + p)
            _reset_and_scan(h_v, vals_v, bufs[p], pn, plo, n, lo, iota,
                            fzero)
            pltpu.async_copy(bufs[p], out_hbm.at[m, pl.ds(v0, HW)], sems[p])
            cur[p] = (n, lo)
        return cur[0] + cur[1]

    drain(0)
    drain(1)


@jax.jit
def _bow(txt, vals):
    txt_t = txt.T.reshape(-1)
    vals_t = vals.T.reshape(-1)
    zchunk = jnp.zeros((HW, B), jnp.float32)
    mesh = plsc.VectorSubcoreMesh(core_axis_name="c", subcore_axis_name="s")
    out = pl.kernel(
        _bow_sc,
        mesh=mesh,
        compiler_params=pltpu.CompilerParams(
            needs_layout_passes=False, use_tc_tiling_on_sc=True),
        out_type=jax.ShapeDtypeStruct((L, DIM, B), jnp.float32),
        scratch_types=[
            pltpu.VMEM((L * B,), jnp.int32),
            pltpu.VMEM((L * B,), jnp.float32),
            pltpu.VMEM((HW, B), jnp.float32),
            pltpu.VMEM((HW, B), jnp.float32),
            pltpu.SemaphoreType.DMA,
            pltpu.SemaphoreType.DMA,
            pltpu.SemaphoreType.DMA,
        ],
    )(txt_t, vals_t, zchunk)
    return out.transpose(2, 0, 1)


def kernel(txt, vals):
    return _bow(txt, vals)
